# baseline (speedup 1.0000x reference)
import functools

import jax
import jax.numpy as jnp
import numpy as np
from jax import lax
from jax.experimental import pallas as pl
from jax.experimental.pallas import tpu as pltpu
from jax.experimental.pallas import tpu_sc as plsc

NS = 16
NV = 4
NF = 48
NWCOL = 320
NOUT = 28
PAD = 32

E_TOT = 1600000
N_TOT = 100000

CH = 512
NBLK = E_TOT // CH
GROUPS = CH // 128
PROWS = CH // 8

HALF_A = 1600
HALF_B = NBLK - HALF_A

SC_CORES = 2
SC_TILES = 16
ACC_ROWS = 100096
TILE_SHARE = ACC_ROWS // SC_TILES

BE = 2560
KB = BE // CH
BN = 512
OUT_STRIDE = 100352
OS_BLK = OUT_STRIDE // 8 // (BN // 8)

_SC_PARAMS = pltpu.CompilerParams(use_tc_tiling_on_sc=False)
_MESH = dict(core_axis_name="c", subcore_axis_name="s")


def _sc_gather(node_attr, dst_idx, c0, nch):
  mesh = plsc.VectorSubcoreMesh(**_MESH)

  @functools.partial(
      pl.kernel, mesh=mesh,
      out_type=jax.ShapeDtypeStruct((nch * PROWS, 128), jnp.float32),
      scratch_types=[
          pltpu.VMEM((GROUPS, 128), jnp.int32),
          pltpu.VMEM((CH, NS), jnp.float32),
          pltpu.VMEM((PROWS, 128), jnp.float32),
          pltpu.SemaphoreType.DMA,
      ],
      compiler_params=_SC_PARAMS,
  )
  def k(tbl_hbm, idx_hbm, out_hbm, idx2_v, rows_v, rows_p, sem):
    wid = lax.axis_index("s") * SC_CORES + lax.axis_index("c")

    def do_chunk(rel_id):
      ebase = (c0 + rel_id) * CH
      for g in range(GROUPS):
        pltpu.sync_copy(idx_hbm.at[pl.ds(ebase + g * 128, 128)], idx2_v.at[g])
      descs = [
          pltpu.make_async_copy(
              tbl_hbm.at[idx2_v.at[g]],
              rows_v.at[pl.ds(g * 128, 128)], sem)
          for g in range(GROUPS)
      ]
      for d in descs:
        d.start()
      for d in descs:
        d.wait()

      @pl.loop(0, PROWS)
      def _(r):
        for l in range(8):
          rows_p[r, pl.ds(l * NS, NS)] = rows_v[l * PROWS + r, pl.ds(0, NS)]

      pltpu.sync_copy(rows_p, out_hbm.at[pl.ds(rel_id * PROWS, PROWS)])

    n_per = nch // 32
    n_extra = nch - 32 * n_per

    @pl.loop(0, n_per)
    def _(kk):
      do_chunk(wid + 32 * kk)

    if n_extra:
      @pl.when(wid < n_extra)
      def _():
        do_chunk(32 * n_per + wid)

  return k(node_attr, dst_idx)


def _tc_dense(ea_t, xsp, sh_t, w1, b1, w2p, b2p, Rm, Tm, Um, Kc, c0, nch):
  blk0 = (c0 * CH) // BE
  nblk = (nch * CH) // BE

  def body(ea, xp, sh, w1r, b1r, w2r, b2r, Rr, Tr, Ur, kc, lo, hi):
    x = xp[...]
    xs = jnp.concatenate(
        [x[64 * kb: 64 * (kb + 1), NS * l: NS * (l + 1)]
         for kb in range(KB) for l in range(8)], axis=0)
    h = jnp.maximum(
        jnp.dot(ea[...].T, w1r[...], preferred_element_type=jnp.float32)
        + b1r[...], 0.0)
    wp = jnp.dot(h.astype(jnp.bfloat16), w2r[...].astype(jnp.bfloat16),
                 preferred_element_type=jnp.float32) + b2r[...]
    xr = jnp.dot(xs, Rr[...], preferred_element_type=jnp.float32)
    pre = jnp.dot((wp * xr).astype(jnp.bfloat16), Tr[...].astype(jnp.bfloat16),
                  preferred_element_type=jnp.float32)
    shf = jnp.dot(sh[...].T, Ur[...], preferred_element_type=jnp.float32)
    tp = pre * shf + kc[...]
    lo[...] = jnp.concatenate(
        [jnp.concatenate(
            [tp[CH * kb + 64 * l: CH * kb + 64 * (l + 1), :NS]
             for l in range(8)], axis=1) for kb in range(KB)], axis=0)
    hi[...] = jnp.concatenate(
        [jnp.concatenate(
            [tp[CH * kb + 64 * l: CH * kb + 64 * (l + 1), NS:]
             for l in range(8)], axis=1) for kb in range(KB)], axis=0)

  full = lambda a, b: pl.BlockSpec((a, b), lambda i: (0, 0))
  return pl.pallas_call(
      body,
      grid=(nblk,),
      in_specs=[
          pl.BlockSpec((NF, BE), lambda i: (0, i + blk0)),
          pl.BlockSpec((KB * PROWS, 128), lambda i: (i, 0)),
          pl.BlockSpec((9, BE), lambda i: (0, i + blk0)),
          full(NF, NF), full(1, NF), full(NF, NWCOL), full(1, NWCOL),
          full(NS, NWCOL), full(NWCOL, PAD), full(9, PAD), full(1, PAD),
      ],
      out_specs=[
          pl.BlockSpec((KB * PROWS, 128), lambda i: (i, 0)),
          pl.BlockSpec((KB * PROWS, 128), lambda i: (i, 0)),
      ],
      out_shape=[
          jax.ShapeDtypeStruct((nch * PROWS, 128), jnp.float32),
          jax.ShapeDtypeStruct((nch * PROWS, 128), jnp.float32),
      ],
  )(ea_t, xsp, sh_t, w1, b1, w2p, b2p, Rm, Tm, Um, Kc)


def _sc_scatter(tp_lo, tp_hi, src_idx, c0, nch):
  mesh = plsc.VectorSubcoreMesh(**_MESH)
  nc0 = nch // 2

  @functools.partial(
      pl.kernel, mesh=mesh,
      out_type=jax.ShapeDtypeStruct((4 * OUT_STRIDE, NS), jnp.float32),
      scratch_types=[
          pltpu.VMEM((GROUPS, 128), jnp.int32),
          pltpu.VMEM((CH, NS), jnp.float32),
          pltpu.VMEM((PROWS, 128), jnp.float32),
          pltpu.VMEM_SHARED((ACC_ROWS, NS), jnp.float32),
      ],
      compiler_params=_SC_PARAMS,
  )
  def k(lo_hbm, hi_hbm, idx_hbm, out_hbm, idx2_v, rows_v, rows_p, acc):
    c = lax.axis_index("c")
    t = lax.axis_index("s")
    zbase = t * TILE_SHARE

    def phase(tp_hbm, p):
      @pl.loop(0, CH)
      def _(i):
        rows_v[i, pl.ds(0, NS)] = jnp.zeros((NS,), jnp.float32)

      for zi in range(12):
        pltpu.sync_copy(rows_v, acc.at[pl.ds(zbase + zi * CH, CH)])
      pltpu.sync_copy(rows_v.at[pl.ds(0, 112)],
                      acc.at[pl.ds(zbase + 12 * CH, 112)])
      plsc.subcore_barrier()

      def do_chunk(rel_id):
        ebase = (c0 + rel_id) * CH
        pltpu.sync_copy(tp_hbm.at[pl.ds(rel_id * PROWS, PROWS)], rows_p)
        for g in range(GROUPS):
          pltpu.sync_copy(idx_hbm.at[pl.ds(ebase + g * 128, 128)],
                          idx2_v.at[g])

        @pl.loop(0, PROWS)
        def _(r):
          for l in range(8):
            rows_v[l * PROWS + r, pl.ds(0, NS)] = rows_p[r, pl.ds(l * NS, NS)]

        for g in range(GROUPS):
          pltpu.sync_copy(rows_v.at[pl.ds(g * 128, 128)],
                          acc.at[idx2_v.at[g]], add=True)

      start = c * nc0
      my_n = nc0 + c * (nch - 2 * nc0)
      n_per = (nch - nc0) // SC_TILES

      @pl.loop(0, n_per)
      def _(kk):
        @pl.when(t + SC_TILES * kk < my_n)
        def _():
          do_chunk(start + t + SC_TILES * kk)

      @pl.when(t + SC_TILES * n_per < my_n)
      def _():
        do_chunk(start + SC_TILES * n_per + t)

      plsc.subcore_barrier()

      obase = (2 * p + c) * OUT_STRIDE + zbase

      @pl.loop(0, 12)
      def _(di):
        pltpu.sync_copy(acc.at[pl.ds(zbase + di * CH, CH)], rows_v)
        pltpu.sync_copy(rows_v, out_hbm.at[pl.ds(obase + di * CH, CH)])

      pltpu.sync_copy(acc.at[pl.ds(zbase + 12 * CH, 112)],
                      rows_v.at[pl.ds(0, 112)])
      pltpu.sync_copy(rows_v.at[pl.ds(0, 112)],
                      out_hbm.at[pl.ds(obase + 12 * CH, 112)])

      @pl.when(t == SC_TILES - 1)
      def _():
        @pl.loop(0, 256)
        def _(i):
          rows_v[i, pl.ds(0, NS)] = jnp.zeros((NS,), jnp.float32)
        pltpu.sync_copy(
            rows_v.at[pl.ds(0, 256)],
            out_hbm.at[pl.ds((2 * p + c) * OUT_STRIDE + ACC_ROWS, 256)])

    phase(lo_hbm, 0)
    plsc.subcore_barrier()
    phase(hi_hbm, 1)

  return k(tp_lo, tp_hi, src_idx)


def _tc_mean(sa, sb, Vm):
  pr = BN // 8

  def body(a00, a01, a10, a11, b00, b01, b10, b11, vv, o):
    s_lo = a00[...] + a01[...] + b00[...] + b01[...]
    s_hi = a10[...] + a11[...] + b10[...] + b11[...]
    lo_u = jnp.concatenate(
        [s_lo[:, NS * l: NS * (l + 1)] for l in range(8)], axis=0)
    hi_u = jnp.concatenate(
        [s_hi[:, NS * l: NS * (l + 1)] for l in range(8)], axis=0)
    lo_n = jnp.dot(lo_u.T, vv[...], preferred_element_type=jnp.float32)
    hi_n = jnp.dot(hi_u.T, vv[...], preferred_element_type=jnp.float32)
    cnt = jnp.maximum(hi_n[12:13, :], 1.0)
    o[...] = jnp.concatenate([lo_n, hi_n[:12, :]], axis=0) / cnt

  def pspec(arr_idx, pc):
    return pl.BlockSpec((pr, 128), lambda i, o=pc * OS_BLK: (i + o, 0))

  return pl.pallas_call(
      body,
      grid=(OS_BLK,),
      in_specs=[pspec(0, 0), pspec(0, 1), pspec(0, 2), pspec(0, 3),
                pspec(1, 0), pspec(1, 1), pspec(1, 2), pspec(1, 3),
                pl.BlockSpec((BN, BN), lambda i: (0, 0))],
      out_specs=pl.BlockSpec((NOUT, BN), lambda i: (0, i)),
      out_shape=jax.ShapeDtypeStruct((NOUT, N_TOT), jnp.float32),
  )(sa, sa, sa, sa, sb, sb, sb, sb, Vm)


def _constants():
  norm = 1.0 / np.sqrt(np.float32(NS))
  perm = np.empty((NWCOL,), np.int64)
  for i in range(NS):
    perm[i * 20: i * 20 + 16] = i * 16 + np.arange(16)
    perm[i * 20 + 16: i * 20 + 20] = NS * NS + i * NV + np.arange(NV)
  Rm = np.zeros((NS, NWCOL), np.float32)
  for i in range(NS):
    Rm[i, i * 20: (i + 1) * 20] = 1.0
  Tm = np.zeros((NWCOL, PAD), np.float32)
  for i in range(NS):
    for cidx in range(16):
      Tm[i * 20 + cidx, cidx] = norm
    for j in range(NV):
      for m in range(3):
        Tm[i * 20 + 16 + j, 16 + 3 * j + m] = norm
  Um = np.zeros((9, PAD), np.float32)
  Um[0, :16] = 1.0
  for j in range(NV):
    for m in range(3):
      Um[1 + m, 16 + 3 * j + m] = 1.0
  Vm = np.zeros((BN, BN), np.float32)
  for l in range(8):
    for r in range(BN // 8):
      Vm[l * (BN // 8) + r, 8 * r + l] = 1.0
  return perm, Rm, Tm, Um, Vm


_PERM, _R, _T, _U, _V = _constants()


def kernel(node_attr, edge_attr, edge_sh, fc_w1, fc_b1, fc_w2, fc_b2,
           edge_index):
  idx32 = edge_index.astype(jnp.int32)
  src, dst = idx32[0], idx32[1]
  w2p = fc_w2[:, _PERM]
  b2p = fc_b2[_PERM].reshape(1, NWCOL)
  b1 = fc_b1.reshape(1, NF)
  Rj, Tj, Uj = jnp.asarray(_R), jnp.asarray(_T), jnp.asarray(_U)
  Vj = jnp.asarray(_V)
  Kc = jnp.zeros((1, PAD), jnp.float32).at[0, NOUT].set(1.0)
  ea_t, sh_t = edge_attr.T, edge_sh.T

  parts = []
  for c0, nch in ((0, HALF_A), (HALF_A, HALF_B)):
    xsp = _sc_gather(node_attr, dst, c0, nch)
    tp_lo, tp_hi = _tc_dense(ea_t, xsp, sh_t, fc_w1, b1, w2p, b2p,
                             Rj, Tj, Uj, Kc, c0, nch)
    parts.append(_sc_scatter(tp_lo, tp_hi, src, c0, nch))
  sa = parts[0].reshape(4 * OUT_STRIDE // 8, 128)
  sb = parts[1].reshape(4 * OUT_STRIDE // 8, 128)
  return _tc_mean(sa, sb, Vj).T


# speedup vs baseline: 1.0011x; 1.0011x over previous
import functools

import jax
import jax.numpy as jnp
import numpy as np
from jax import lax
from jax.experimental import pallas as pl
from jax.experimental.pallas import tpu as pltpu
from jax.experimental.pallas import tpu_sc as plsc

NS = 16
NV = 4
NF = 48
NWCOL = 320
NOUT = 28
PAD = 32

E_TOT = 1600000
N_TOT = 100000

CH = 512
NBLK = E_TOT // CH
GROUPS = CH // 128
PROWS = CH // 8

HALF_A = 1600
HALF_B = NBLK - HALF_A

SC_CORES = 2
SC_TILES = 16
ACC_ROWS = 100096
TILE_SHARE = ACC_ROWS // SC_TILES

BE = 2560
KB = BE // CH
BN = 512
OUT_STRIDE = 100352
OS_BLK = OUT_STRIDE // 8 // (BN // 8)

_SC_PARAMS = pltpu.CompilerParams(use_tc_tiling_on_sc=False)
_MESH = dict(core_axis_name="c", subcore_axis_name="s")


def _sc_gather(node_attr, dst_idx, c0, nch):
  mesh = plsc.VectorSubcoreMesh(**_MESH)

  @functools.partial(
      pl.kernel, mesh=mesh,
      out_type=jax.ShapeDtypeStruct((nch * PROWS, 128), jnp.float32),
      scratch_types=[
          pltpu.VMEM((GROUPS, 128), jnp.int32),
          pltpu.VMEM((CH, NS), jnp.float32),
          pltpu.VMEM((PROWS, 128), jnp.float32),
          pltpu.SemaphoreType.DMA,
      ],
      compiler_params=_SC_PARAMS,
  )
  def k(tbl_hbm, idx_hbm, out_hbm, idx2_v, rows_v, rows_p, sem):
    wid = lax.axis_index("s") * SC_CORES + lax.axis_index("c")

    def do_chunk(rel_id):
      ebase = (c0 + rel_id) * CH
      for g in range(GROUPS):
        pltpu.sync_copy(idx_hbm.at[pl.ds(ebase + g * 128, 128)], idx2_v.at[g])
      descs = [
          pltpu.make_async_copy(
              tbl_hbm.at[idx2_v.at[g]],
              rows_v.at[pl.ds(g * 128, 128)], sem)
          for g in range(GROUPS)
      ]
      for d in descs:
        d.start()
      for d in descs:
        d.wait()

      @pl.loop(0, PROWS)
      def _(r):
        for l in range(8):
          rows_p[r, pl.ds(l * NS, NS)] = rows_v[l * PROWS + r, pl.ds(0, NS)]

      pltpu.sync_copy(rows_p, out_hbm.at[pl.ds(rel_id * PROWS, PROWS)])

    n_per = nch // 32
    n_extra = nch - 32 * n_per

    @pl.loop(0, n_per)
    def _(kk):
      do_chunk(wid + 32 * kk)

    if n_extra:
      @pl.when(wid < n_extra)
      def _():
        do_chunk(32 * n_per + wid)

  return k(node_attr, dst_idx)


def _tc_dense(ea_t, xsp, sh_t, w1, b1, w2p, b2p, Rm, Tm, Um, Kc, c0, nch):
  blk0 = (c0 * CH) // BE
  nblk = (nch * CH) // BE

  def body(ea, xp, sh, w1r, b1r, w2r, b2r, Rr, Tr, Ur, kc, lo, hi):
    x = xp[...]
    xs = jnp.concatenate(
        [x[64 * kb: 64 * (kb + 1), NS * l: NS * (l + 1)]
         for kb in range(KB) for l in range(8)], axis=0)
    h = jnp.maximum(
        jnp.dot(ea[...].T, w1r[...], preferred_element_type=jnp.float32)
        + b1r[...], 0.0)
    wp = jnp.dot(h, w2r[...], preferred_element_type=jnp.float32) + b2r[...]
    xr = jnp.dot(xs, Rr[...], preferred_element_type=jnp.float32)
    pre = jnp.dot(wp * xr, Tr[...], preferred_element_type=jnp.float32)
    shf = jnp.dot(sh[...].T, Ur[...], preferred_element_type=jnp.float32)
    tp = pre * shf + kc[...]
    lo[...] = jnp.concatenate(
        [jnp.concatenate(
            [tp[CH * kb + 64 * l: CH * kb + 64 * (l + 1), :NS]
             for l in range(8)], axis=1) for kb in range(KB)], axis=0)
    hi[...] = jnp.concatenate(
        [jnp.concatenate(
            [tp[CH * kb + 64 * l: CH * kb + 64 * (l + 1), NS:]
             for l in range(8)], axis=1) for kb in range(KB)], axis=0)

  full = lambda a, b: pl.BlockSpec((a, b), lambda i: (0, 0))
  return pl.pallas_call(
      body,
      grid=(nblk,),
      in_specs=[
          pl.BlockSpec((NF, BE), lambda i: (0, i + blk0)),
          pl.BlockSpec((KB * PROWS, 128), lambda i: (i, 0)),
          pl.BlockSpec((9, BE), lambda i: (0, i + blk0)),
          full(NF, NF), full(1, NF), full(NF, NWCOL), full(1, NWCOL),
          full(NS, NWCOL), full(NWCOL, PAD), full(9, PAD), full(1, PAD),
      ],
      out_specs=[
          pl.BlockSpec((KB * PROWS, 128), lambda i: (i, 0)),
          pl.BlockSpec((KB * PROWS, 128), lambda i: (i, 0)),
      ],
      out_shape=[
          jax.ShapeDtypeStruct((nch * PROWS, 128), jnp.float32),
          jax.ShapeDtypeStruct((nch * PROWS, 128), jnp.float32),
      ],
  )(ea_t, xsp, sh_t, w1, b1, w2p, b2p, Rm, Tm, Um, Kc)


def _sc_scatter(tp_lo, tp_hi, src_idx, c0, nch):
  mesh = plsc.VectorSubcoreMesh(**_MESH)
  nc0 = nch // 2

  @functools.partial(
      pl.kernel, mesh=mesh,
      out_type=jax.ShapeDtypeStruct((4 * OUT_STRIDE, NS), jnp.float32),
      scratch_types=[
          pltpu.VMEM((GROUPS, 128), jnp.int32),
          pltpu.VMEM((CH, NS), jnp.float32),
          pltpu.VMEM((PROWS, 128), jnp.float32),
          pltpu.VMEM_SHARED((ACC_ROWS, NS), jnp.float32),
      ],
      compiler_params=_SC_PARAMS,
  )
  def k(lo_hbm, hi_hbm, idx_hbm, out_hbm, idx2_v, rows_v, rows_p, acc):
    c = lax.axis_index("c")
    t = lax.axis_index("s")
    zbase = t * TILE_SHARE

    def phase(tp_hbm, p):
      @pl.loop(0, CH)
      def _(i):
        rows_v[i, pl.ds(0, NS)] = jnp.zeros((NS,), jnp.float32)

      for zi in range(12):
        pltpu.sync_copy(rows_v, acc.at[pl.ds(zbase + zi * CH, CH)])
      pltpu.sync_copy(rows_v.at[pl.ds(0, 112)],
                      acc.at[pl.ds(zbase + 12 * CH, 112)])
      plsc.subcore_barrier()

      def do_chunk(rel_id):
        ebase = (c0 + rel_id) * CH
        pltpu.sync_copy(tp_hbm.at[pl.ds(rel_id * PROWS, PROWS)], rows_p)
        for g in range(GROUPS):
          pltpu.sync_copy(idx_hbm.at[pl.ds(ebase + g * 128, 128)],
                          idx2_v.at[g])

        @pl.loop(0, PROWS)
        def _(r):
          for l in range(8):
            rows_v[l * PROWS + r, pl.ds(0, NS)] = rows_p[r, pl.ds(l * NS, NS)]

        for g in range(GROUPS):
          pltpu.sync_copy(rows_v.at[pl.ds(g * 128, 128)],
                          acc.at[idx2_v.at[g]], add=True)

      start = c * nc0
      my_n = nc0 + c * (nch - 2 * nc0)
      n_per = (nch - nc0) // SC_TILES

      @pl.loop(0, n_per)
      def _(kk):
        @pl.when(t + SC_TILES * kk < my_n)
        def _():
          do_chunk(start + t + SC_TILES * kk)

      @pl.when(t + SC_TILES * n_per < my_n)
      def _():
        do_chunk(start + SC_TILES * n_per + t)

      plsc.subcore_barrier()

      obase = (2 * p + c) * OUT_STRIDE + zbase

      @pl.loop(0, 12)
      def _(di):
        pltpu.sync_copy(acc.at[pl.ds(zbase + di * CH, CH)], rows_v)
        pltpu.sync_copy(rows_v, out_hbm.at[pl.ds(obase + di * CH, CH)])

      pltpu.sync_copy(acc.at[pl.ds(zbase + 12 * CH, 112)],
                      rows_v.at[pl.ds(0, 112)])
      pltpu.sync_copy(rows_v.at[pl.ds(0, 112)],
                      out_hbm.at[pl.ds(obase + 12 * CH, 112)])

      @pl.when(t == SC_TILES - 1)
      def _():
        @pl.loop(0, 256)
        def _(i):
          rows_v[i, pl.ds(0, NS)] = jnp.zeros((NS,), jnp.float32)
        pltpu.sync_copy(
            rows_v.at[pl.ds(0, 256)],
            out_hbm.at[pl.ds((2 * p + c) * OUT_STRIDE + ACC_ROWS, 256)])

    phase(lo_hbm, 0)
    plsc.subcore_barrier()
    phase(hi_hbm, 1)

  return k(tp_lo, tp_hi, src_idx)


def _tc_mean(sa, sb, Vm):
  pr = BN // 8

  def body(a00, a01, a10, a11, b00, b01, b10, b11, vv, o):
    s_lo = a00[...] + a01[...] + b00[...] + b01[...]
    s_hi = a10[...] + a11[...] + b10[...] + b11[...]
    lo_u = jnp.concatenate(
        [s_lo[:, NS * l: NS * (l + 1)] for l in range(8)], axis=0)
    hi_u = jnp.concatenate(
        [s_hi[:, NS * l: NS * (l + 1)] for l in range(8)], axis=0)
    lo_n = jnp.dot(lo_u.T, vv[...], preferred_element_type=jnp.float32)
    hi_n = jnp.dot(hi_u.T, vv[...], preferred_element_type=jnp.float32)
    cnt = jnp.maximum(hi_n[12:13, :], 1.0)
    o[...] = jnp.concatenate([lo_n, hi_n[:12, :]], axis=0) / cnt

  def pspec(arr_idx, pc):
    return pl.BlockSpec((pr, 128), lambda i, o=pc * OS_BLK: (i + o, 0))

  return pl.pallas_call(
      body,
      grid=(OS_BLK,),
      in_specs=[pspec(0, 0), pspec(0, 1), pspec(0, 2), pspec(0, 3),
                pspec(1, 0), pspec(1, 1), pspec(1, 2), pspec(1, 3),
                pl.BlockSpec((BN, BN), lambda i: (0, 0))],
      out_specs=pl.BlockSpec((NOUT, BN), lambda i: (0, i)),
      out_shape=jax.ShapeDtypeStruct((NOUT, N_TOT), jnp.float32),
  )(sa, sa, sa, sa, sb, sb, sb, sb, Vm)


def _constants():
  norm = 1.0 / np.sqrt(np.float32(NS))
  perm = np.empty((NWCOL,), np.int64)
  for i in range(NS):
    perm[i * 20: i * 20 + 16] = i * 16 + np.arange(16)
    perm[i * 20 + 16: i * 20 + 20] = NS * NS + i * NV + np.arange(NV)
  Rm = np.zeros((NS, NWCOL), np.float32)
  for i in range(NS):
    Rm[i, i * 20: (i + 1) * 20] = 1.0
  Tm = np.zeros((NWCOL, PAD), np.float32)
  for i in range(NS):
    for cidx in range(16):
      Tm[i * 20 + cidx, cidx] = norm
    for j in range(NV):
      for m in range(3):
        Tm[i * 20 + 16 + j, 16 + 3 * j + m] = norm
  Um = np.zeros((9, PAD), np.float32)
  Um[0, :16] = 1.0
  for j in range(NV):
    for m in range(3):
      Um[1 + m, 16 + 3 * j + m] = 1.0
  Vm = np.zeros((BN, BN), np.float32)
  for l in range(8):
    for r in range(BN // 8):
      Vm[l * (BN // 8) + r, 8 * r + l] = 1.0
  return perm, Rm, Tm, Um, Vm


_PERM, _R, _T, _U, _V = _constants()


def kernel(node_attr, edge_attr, edge_sh, fc_w1, fc_b1, fc_w2, fc_b2,
           edge_index):
  idx32 = edge_index.astype(jnp.int32)
  src, dst = idx32[0], idx32[1]
  w2p = fc_w2[:, _PERM]
  b2p = fc_b2[_PERM].reshape(1, NWCOL)
  b1 = fc_b1.reshape(1, NF)
  Rj, Tj, Uj = jnp.asarray(_R), jnp.asarray(_T), jnp.asarray(_U)
  Vj = jnp.asarray(_V)
  Kc = jnp.zeros((1, PAD), jnp.float32).at[0, NOUT].set(1.0)
  ea_t, sh_t = edge_attr.T, edge_sh.T

  parts = []
  for c0, nch in ((0, HALF_A), (HALF_A, HALF_B)):
    xsp = _sc_gather(node_attr, dst, c0, nch)
    tp_lo, tp_hi = _tc_dense(ea_t, xsp, sh_t, fc_w1, b1, w2p, b2p,
                             Rj, Tj, Uj, Kc, c0, nch)
    parts.append(_sc_scatter(tp_lo, tp_hi, src, c0, nch))
  sa = parts[0].reshape(4 * OUT_STRIDE // 8, 128)
  sb = parts[1].reshape(4 * OUT_STRIDE // 8, 128)
  return _tc_mean(sa, sb, Vj).T


# speedup vs baseline: 1.1859x; 1.1846x over previous
import functools

import jax
import jax.numpy as jnp
import numpy as np
from jax import lax
from jax.experimental import pallas as pl
from jax.experimental.pallas import tpu as pltpu
from jax.experimental.pallas import tpu_sc as plsc

NS = 16
NV = 4
NF = 48
NWCOL = 320
NOUT = 28
PAD = 32

E_TOT = 1600000
N_TOT = 100000

CH = 512
NBLK = E_TOT // CH
GROUPS = CH // 128
PROWS = CH // 8

SPLITS = 5
SP_CH = NBLK // SPLITS

SC_CORES = 2
SC_TILES = 16
ACC_ROWS = 100096
TILE_SHARE = ACC_ROWS // SC_TILES

BE = 2560
KB = BE // CH
BN = 512
OUT_STRIDE = 100352
OS_BLK = OUT_STRIDE // 8 // (BN // 8)

_SC_PARAMS = pltpu.CompilerParams(use_tc_tiling_on_sc=False)
_MESH = dict(core_axis_name="c", subcore_axis_name="s")


def _sc_gather(node_attr, dst_idx, c0, nch):
  mesh = plsc.VectorSubcoreMesh(**_MESH)

  @functools.partial(
      pl.kernel, mesh=mesh,
      out_type=jax.ShapeDtypeStruct((nch * PROWS, 128), jnp.float32),
      scratch_types=[
          pltpu.VMEM((GROUPS, 128), jnp.int32),
          pltpu.VMEM((CH, NS), jnp.float32),
          pltpu.VMEM((PROWS, 128), jnp.float32),
          pltpu.SemaphoreType.DMA,
      ],
      compiler_params=_SC_PARAMS,
  )
  def k(tbl_hbm, idx_hbm, out_hbm, idx2_v, rows_v, rows_p, sem):
    wid = lax.axis_index("s") * SC_CORES + lax.axis_index("c")

    def do_chunk(rel_id):
      ebase = (c0 + rel_id) * CH
      for g in range(GROUPS):
        pltpu.sync_copy(idx_hbm.at[pl.ds(ebase + g * 128, 128)], idx2_v.at[g])
      descs = [
          pltpu.make_async_copy(
              tbl_hbm.at[idx2_v.at[g]],
              rows_v.at[pl.ds(g * 128, 128)], sem)
          for g in range(GROUPS)
      ]
      for d in descs:
        d.start()
      for d in descs:
        d.wait()

      @pl.loop(0, PROWS)
      def _(r):
        for l in range(8):
          rows_p[r, pl.ds(l * NS, NS)] = rows_v[l * PROWS + r, pl.ds(0, NS)]

      pltpu.sync_copy(rows_p, out_hbm.at[pl.ds(rel_id * PROWS, PROWS)])

    n_per = nch // 32
    n_extra = nch - 32 * n_per

    @pl.loop(0, n_per)
    def _(kk):
      do_chunk(wid + 32 * kk)

    if n_extra:
      @pl.when(wid < n_extra)
      def _():
        do_chunk(32 * n_per + wid)

  return k(node_attr, dst_idx)


def _tc_dense(ea_t, xsp, sh_t, w1, b1, w2p, b2p, Rm, Tm, Um, Kc, c0, nch):
  blk0 = (c0 * CH) // BE
  nblk = (nch * CH) // BE

  def body(ea, xp, sh, w1r, b1r, w2r, b2r, Rr, Tr, Ur, kc, lo, hi):
    x = xp[...]
    xs = jnp.concatenate(
        [x[64 * kb: 64 * (kb + 1), NS * l: NS * (l + 1)]
         for kb in range(KB) for l in range(8)], axis=0)
    h = jnp.maximum(
        jnp.dot(ea[...].T, w1r[...], preferred_element_type=jnp.float32)
        + b1r[...], 0.0)
    wp = jnp.dot(h, w2r[...], preferred_element_type=jnp.float32) + b2r[...]
    xr = jnp.dot(xs, Rr[...], preferred_element_type=jnp.float32)
    pre = jnp.dot(wp * xr, Tr[...], preferred_element_type=jnp.float32)
    shf = jnp.dot(sh[...].T, Ur[...], preferred_element_type=jnp.float32)
    tp = pre * shf + kc[...]
    lo[...] = jnp.concatenate(
        [jnp.concatenate(
            [tp[CH * kb + 64 * l: CH * kb + 64 * (l + 1), :NS]
             for l in range(8)], axis=1) for kb in range(KB)], axis=0)
    hi[...] = jnp.concatenate(
        [jnp.concatenate(
            [tp[CH * kb + 64 * l: CH * kb + 64 * (l + 1), NS:]
             for l in range(8)], axis=1) for kb in range(KB)], axis=0)

  full = lambda a, b: pl.BlockSpec((a, b), lambda i: (0, 0))
  return pl.pallas_call(
      body,
      grid=(nblk,),
      in_specs=[
          pl.BlockSpec((NF, BE), lambda i: (0, i + blk0)),
          pl.BlockSpec((KB * PROWS, 128), lambda i: (i, 0)),
          pl.BlockSpec((9, BE), lambda i: (0, i + blk0)),
          full(NF, NF), full(1, NF), full(NF, NWCOL), full(1, NWCOL),
          full(NS, NWCOL), full(NWCOL, PAD), full(9, PAD), full(1, PAD),
      ],
      out_specs=[
          pl.BlockSpec((KB * PROWS, 128), lambda i: (i, 0)),
          pl.BlockSpec((KB * PROWS, 128), lambda i: (i, 0)),
      ],
      out_shape=[
          jax.ShapeDtypeStruct((nch * PROWS, 128), jnp.float32),
          jax.ShapeDtypeStruct((nch * PROWS, 128), jnp.float32),
      ],
  )(ea_t, xsp, sh_t, w1, b1, w2p, b2p, Rm, Tm, Um, Kc)


def _sc_scatter(tp_lo, tp_hi, src_idx, c0, nch):
  mesh = plsc.VectorSubcoreMesh(**_MESH)
  nc0 = nch // 2

  @functools.partial(
      pl.kernel, mesh=mesh,
      out_type=jax.ShapeDtypeStruct((4 * OUT_STRIDE, NS), jnp.float32),
      scratch_types=[
          pltpu.VMEM((GROUPS, 128), jnp.int32),
          pltpu.VMEM((CH, NS), jnp.float32),
          pltpu.VMEM((PROWS, 128), jnp.float32),
          pltpu.VMEM_SHARED((ACC_ROWS, NS), jnp.float32),
      ],
      compiler_params=_SC_PARAMS,
  )
  def k(lo_hbm, hi_hbm, idx_hbm, out_hbm, idx2_v, rows_v, rows_p, acc):
    c = lax.axis_index("c")
    t = lax.axis_index("s")
    zbase = t * TILE_SHARE

    def phase(tp_hbm, p):
      @pl.loop(0, CH)
      def _(i):
        rows_v[i, pl.ds(0, NS)] = jnp.zeros((NS,), jnp.float32)

      for zi in range(12):
        pltpu.sync_copy(rows_v, acc.at[pl.ds(zbase + zi * CH, CH)])
      pltpu.sync_copy(rows_v.at[pl.ds(0, 112)],
                      acc.at[pl.ds(zbase + 12 * CH, 112)])
      plsc.subcore_barrier()

      def do_chunk(rel_id):
        ebase = (c0 + rel_id) * CH
        pltpu.sync_copy(tp_hbm.at[pl.ds(rel_id * PROWS, PROWS)], rows_p)
        for g in range(GROUPS):
          pltpu.sync_copy(idx_hbm.at[pl.ds(ebase + g * 128, 128)],
                          idx2_v.at[g])

        @pl.loop(0, PROWS)
        def _(r):
          for l in range(8):
            rows_v[l * PROWS + r, pl.ds(0, NS)] = rows_p[r, pl.ds(l * NS, NS)]

        for g in range(GROUPS):
          pltpu.sync_copy(rows_v.at[pl.ds(g * 128, 128)],
                          acc.at[idx2_v.at[g]], add=True)

      start = c * nc0
      my_n = nc0 + c * (nch - 2 * nc0)
      n_per = (nch - nc0) // SC_TILES

      @pl.loop(0, n_per)
      def _(kk):
        @pl.when(t + SC_TILES * kk < my_n)
        def _():
          do_chunk(start + t + SC_TILES * kk)

      @pl.when(t + SC_TILES * n_per < my_n)
      def _():
        do_chunk(start + SC_TILES * n_per + t)

      plsc.subcore_barrier()

      obase = (2 * p + c) * OUT_STRIDE + zbase

      @pl.loop(0, 12)
      def _(di):
        pltpu.sync_copy(acc.at[pl.ds(zbase + di * CH, CH)], rows_v)
        pltpu.sync_copy(rows_v, out_hbm.at[pl.ds(obase + di * CH, CH)])

      pltpu.sync_copy(acc.at[pl.ds(zbase + 12 * CH, 112)],
                      rows_v.at[pl.ds(0, 112)])
      pltpu.sync_copy(rows_v.at[pl.ds(0, 112)],
                      out_hbm.at[pl.ds(obase + 12 * CH, 112)])

      @pl.when(t == SC_TILES - 1)
      def _():
        @pl.loop(0, 256)
        def _(i):
          rows_v[i, pl.ds(0, NS)] = jnp.zeros((NS,), jnp.float32)
        pltpu.sync_copy(
            rows_v.at[pl.ds(0, 256)],
            out_hbm.at[pl.ds((2 * p + c) * OUT_STRIDE + ACC_ROWS, 256)])

    phase(lo_hbm, 0)
    plsc.subcore_barrier()
    phase(hi_hbm, 1)

  return k(tp_lo, tp_hi, src_idx)


def _tc_mean(sa, Vm):
  pr = BN // 8
  n_arr = len(sa)

  def body(*refs):
    ins, vv, o = refs[:4 * n_arr], refs[4 * n_arr], refs[4 * n_arr + 1]
    s_lo = sum(ins[4 * a][...] + ins[4 * a + 1][...] for a in range(n_arr))
    s_hi = sum(ins[4 * a + 2][...] + ins[4 * a + 3][...] for a in range(n_arr))
    lo_u = jnp.concatenate(
        [s_lo[:, NS * l: NS * (l + 1)] for l in range(8)], axis=0)
    hi_u = jnp.concatenate(
        [s_hi[:, NS * l: NS * (l + 1)] for l in range(8)], axis=0)
    lo_n = jnp.dot(lo_u.T, vv[...], preferred_element_type=jnp.float32)
    hi_n = jnp.dot(hi_u.T, vv[...], preferred_element_type=jnp.float32)
    cnt = jnp.maximum(hi_n[12:13, :], 1.0)
    o[...] = jnp.concatenate([lo_n, hi_n[:12, :]], axis=0) / cnt

  def pspec(pc):
    return pl.BlockSpec((pr, 128), lambda i, o=pc * OS_BLK: (i + o, 0))

  return pl.pallas_call(
      body,
      grid=(OS_BLK,),
      in_specs=[pspec(pc) for _ in range(n_arr) for pc in range(4)]
               + [pl.BlockSpec((BN, BN), lambda i: (0, 0))],
      out_specs=pl.BlockSpec((NOUT, BN), lambda i: (0, i)),
      out_shape=jax.ShapeDtypeStruct((NOUT, N_TOT), jnp.float32),
  )(*[a for a in sa for _ in range(4)], Vm)


def _constants():
  norm = 1.0 / np.sqrt(np.float32(NS))
  perm = np.empty((NWCOL,), np.int64)
  for i in range(NS):
    perm[i * 20: i * 20 + 16] = i * 16 + np.arange(16)
    perm[i * 20 + 16: i * 20 + 20] = NS * NS + i * NV + np.arange(NV)
  Rm = np.zeros((NS, NWCOL), np.float32)
  for i in range(NS):
    Rm[i, i * 20: (i + 1) * 20] = 1.0
  Tm = np.zeros((NWCOL, PAD), np.float32)
  for i in range(NS):
    for cidx in range(16):
      Tm[i * 20 + cidx, cidx] = norm
    for j in range(NV):
      for m in range(3):
        Tm[i * 20 + 16 + j, 16 + 3 * j + m] = norm
  Um = np.zeros((9, PAD), np.float32)
  Um[0, :16] = 1.0
  for j in range(NV):
    for m in range(3):
      Um[1 + m, 16 + 3 * j + m] = 1.0
  Vm = np.zeros((BN, BN), np.float32)
  for l in range(8):
    for r in range(BN // 8):
      Vm[l * (BN // 8) + r, 8 * r + l] = 1.0
  return perm, Rm, Tm, Um, Vm


_PERM, _R, _T, _U, _V = _constants()


def kernel(node_attr, edge_attr, edge_sh, fc_w1, fc_b1, fc_w2, fc_b2,
           edge_index):
  idx32 = edge_index.astype(jnp.int32)
  src, dst = idx32[0], idx32[1]
  w2p = fc_w2[:, _PERM]
  b2p = fc_b2[_PERM].reshape(1, NWCOL)
  b1 = fc_b1.reshape(1, NF)
  Rj, Tj, Uj = jnp.asarray(_R), jnp.asarray(_T), jnp.asarray(_U)
  Vj = jnp.asarray(_V)
  Kc = jnp.zeros((1, PAD), jnp.float32).at[0, NOUT].set(1.0)
  ea_t, sh_t = edge_attr.T, edge_sh.T

  parts = []
  for s in range(SPLITS):
    c0, nch = s * SP_CH, SP_CH
    xsp = _sc_gather(node_attr, dst, c0, nch)
    tp_lo, tp_hi = _tc_dense(ea_t, xsp, sh_t, fc_w1, b1, w2p, b2p,
                             Rj, Tj, Uj, Kc, c0, nch)
    parts.append(_sc_scatter(tp_lo, tp_hi, src, c0, nch))
  packed = [p.reshape(4 * OUT_STRIDE // 8, 128) for p in parts]
  return _tc_mean(packed, Vj).T


# speedup vs baseline: 1.2046x; 1.0158x over previous
import functools

import jax
import jax.numpy as jnp
import numpy as np
from jax import lax
from jax.experimental import pallas as pl
from jax.experimental.pallas import tpu as pltpu
from jax.experimental.pallas import tpu_sc as plsc

NS = 16
NV = 4
NF = 48
NWCOL = 320
NOUT = 28
PAD = 32

E_TOT = 1600000
N_TOT = 100000

CH = 512
NBLK = E_TOT // CH
GROUPS = CH // 128
PROWS = CH // 8

SPLITS = 5
SP_CH = NBLK // SPLITS

SC_CORES = 2
SC_TILES = 16
ACC_ROWS = 100096
TILE_SHARE = ACC_ROWS // SC_TILES

BE = 2560
KB = BE // CH
BN = 512
OUT_STRIDE = 100352
OS_BLK = OUT_STRIDE // 8 // (BN // 8)

_SC_PARAMS = pltpu.CompilerParams(use_tc_tiling_on_sc=False)
_MESH = dict(core_axis_name="c", subcore_axis_name="s")


def _sc_gather(node_attr, dst_idx, c0, nch):
  mesh = plsc.VectorSubcoreMesh(**_MESH)

  @functools.partial(
      pl.kernel, mesh=mesh,
      out_type=jax.ShapeDtypeStruct((nch * PROWS, 128), jnp.float32),
      scratch_types=[
          pltpu.VMEM((GROUPS, 128), jnp.int32),
          pltpu.VMEM((CH, NS), jnp.float32),
          pltpu.VMEM((PROWS, 128), jnp.float32),
          pltpu.SemaphoreType.DMA,
      ],
      compiler_params=_SC_PARAMS,
  )
  def k(tbl_hbm, idx_hbm, out_hbm, idx2_v, rows_v, rows_p, sem):
    wid = lax.axis_index("s") * SC_CORES + lax.axis_index("c")

    def do_chunk(rel_id):
      ebase = (c0 + rel_id) * CH
      for g in range(GROUPS):
        pltpu.sync_copy(idx_hbm.at[pl.ds(ebase + g * 128, 128)], idx2_v.at[g])
      descs = [
          pltpu.make_async_copy(
              tbl_hbm.at[idx2_v.at[g]],
              rows_v.at[pl.ds(g * 128, 128)], sem)
          for g in range(GROUPS)
      ]
      for d in descs:
        d.start()
      for d in descs:
        d.wait()

      @pl.loop(0, PROWS)
      def _(r):
        for l in range(8):
          rows_p[r, pl.ds(l * NS, NS)] = rows_v[l * PROWS + r, pl.ds(0, NS)]

      pltpu.sync_copy(rows_p, out_hbm.at[pl.ds(rel_id * PROWS, PROWS)])

    n_per = nch // 32
    n_extra = nch - 32 * n_per

    @pl.loop(0, n_per)
    def _(kk):
      do_chunk(wid + 32 * kk)

    if n_extra:
      @pl.when(wid < n_extra)
      def _():
        do_chunk(32 * n_per + wid)

  return k(node_attr, dst_idx)


def _tc_dense(ea_t, xsp, sh_t, w1, b1, w2p, b2p, Rm, Tm, Um, Kc, c0, nch):
  blk0 = (c0 * CH) // BE
  nblk = (nch * CH) // BE

  def body(ea, xp, sh, w1r, b1r, w2r, b2r, Rr, Tr, Ur, kc, lo, hi):
    x = xp[...]
    xs = jnp.concatenate(
        [x[64 * kb: 64 * (kb + 1), NS * l: NS * (l + 1)]
         for kb in range(KB) for l in range(8)], axis=0)
    h = jnp.maximum(
        jnp.dot(ea[...].T, w1r[...], preferred_element_type=jnp.float32)
        + b1r[...], 0.0)
    wp = jnp.dot(h, w2r[...], preferred_element_type=jnp.float32) + b2r[...]
    xr = jnp.dot(xs, Rr[...], preferred_element_type=jnp.float32)
    pre = jnp.dot(wp * xr, Tr[...], preferred_element_type=jnp.float32)
    shf = jnp.dot(sh[...].T, Ur[...], preferred_element_type=jnp.float32)
    tp = pre * shf + kc[...]
    lo[...] = jnp.concatenate(
        [jnp.concatenate(
            [tp[CH * kb + 64 * l: CH * kb + 64 * (l + 1), :NS]
             for l in range(8)], axis=1) for kb in range(KB)], axis=0)
    hi[...] = jnp.concatenate(
        [jnp.concatenate(
            [tp[CH * kb + 64 * l: CH * kb + 64 * (l + 1), NS:]
             for l in range(8)], axis=1) for kb in range(KB)], axis=0)

  full = lambda a, b: pl.BlockSpec((a, b), lambda i: (0, 0))
  return pl.pallas_call(
      body,
      grid=(nblk,),
      in_specs=[
          pl.BlockSpec((NF, BE), lambda i: (0, i + blk0)),
          pl.BlockSpec((KB * PROWS, 128), lambda i: (i, 0)),
          pl.BlockSpec((9, BE), lambda i: (0, i + blk0)),
          full(NF, NF), full(1, NF), full(NF, NWCOL), full(1, NWCOL),
          full(NS, NWCOL), full(NWCOL, PAD), full(9, PAD), full(1, PAD),
      ],
      out_specs=[
          pl.BlockSpec((KB * PROWS, 128), lambda i: (i, 0)),
          pl.BlockSpec((KB * PROWS, 128), lambda i: (i, 0)),
      ],
      out_shape=[
          jax.ShapeDtypeStruct((nch * PROWS, 128), jnp.float32),
          jax.ShapeDtypeStruct((nch * PROWS, 128), jnp.float32),
      ],
  )(ea_t, xsp, sh_t, w1, b1, w2p, b2p, Rm, Tm, Um, Kc)


def _sc_scatter(tp_lo, tp_hi, src_idx, c0, nch):
  mesh = plsc.VectorSubcoreMesh(**_MESH)

  @functools.partial(
      pl.kernel, mesh=mesh,
      out_type=jax.ShapeDtypeStruct((2 * OUT_STRIDE, NS), jnp.float32),
      scratch_types=[
          pltpu.VMEM((GROUPS, 128), jnp.int32),
          pltpu.VMEM((CH, NS), jnp.float32),
          pltpu.VMEM((PROWS, 128), jnp.float32),
          pltpu.VMEM_SHARED((ACC_ROWS, NS), jnp.float32),
      ],
      compiler_params=_SC_PARAMS,
  )
  def k(lo_hbm, hi_hbm, idx_hbm, out_hbm, idx2_v, rows_v, rows_p, acc):
    c = lax.axis_index("c")
    t = lax.axis_index("s")
    zbase = t * TILE_SHARE

    @pl.loop(0, CH)
    def _(i):
      rows_v[i, pl.ds(0, NS)] = jnp.zeros((NS,), jnp.float32)

    for zi in range(12):
      pltpu.sync_copy(rows_v, acc.at[pl.ds(zbase + zi * CH, CH)])
    pltpu.sync_copy(rows_v.at[pl.ds(0, 112)],
                    acc.at[pl.ds(zbase + 12 * CH, 112)])
    plsc.subcore_barrier()

    def do_chunk(tp_hbm, rel_id):
      ebase = (c0 + rel_id) * CH
      pltpu.sync_copy(tp_hbm.at[pl.ds(rel_id * PROWS, PROWS)], rows_p)
      for g in range(GROUPS):
        pltpu.sync_copy(idx_hbm.at[pl.ds(ebase + g * 128, 128)],
                        idx2_v.at[g])

      @pl.loop(0, PROWS)
      def _(r):
        for l in range(8):
          rows_v[l * PROWS + r, pl.ds(0, NS)] = rows_p[r, pl.ds(l * NS, NS)]

      for g in range(GROUPS):
        pltpu.sync_copy(rows_v.at[pl.ds(g * 128, 128)],
                        acc.at[idx2_v.at[g]], add=True)

    n_per = nch // SC_TILES
    n_extra = nch - SC_TILES * n_per

    def run(tp_hbm):
      @pl.loop(0, n_per)
      def _(kk):
        do_chunk(tp_hbm, t + SC_TILES * kk)
      if n_extra:
        @pl.when(t < n_extra)
        def _():
          do_chunk(tp_hbm, SC_TILES * n_per + t)

    @pl.when(c == 0)
    def _():
      run(lo_hbm)

    @pl.when(c == 1)
    def _():
      run(hi_hbm)

    plsc.subcore_barrier()

    obase = c * OUT_STRIDE + zbase

    @pl.loop(0, 12)
    def _(di):
      pltpu.sync_copy(acc.at[pl.ds(zbase + di * CH, CH)], rows_v)
      pltpu.sync_copy(rows_v, out_hbm.at[pl.ds(obase + di * CH, CH)])

    pltpu.sync_copy(acc.at[pl.ds(zbase + 12 * CH, 112)],
                    rows_v.at[pl.ds(0, 112)])
    pltpu.sync_copy(rows_v.at[pl.ds(0, 112)],
                    out_hbm.at[pl.ds(obase + 12 * CH, 112)])

    @pl.when(t == SC_TILES - 1)
    def _():
      @pl.loop(0, 256)
      def _(i):
        rows_v[i, pl.ds(0, NS)] = jnp.zeros((NS,), jnp.float32)
      pltpu.sync_copy(
          rows_v.at[pl.ds(0, 256)],
          out_hbm.at[pl.ds(c * OUT_STRIDE + ACC_ROWS, 256)])

  return k(tp_lo, tp_hi, src_idx)


def _tc_mean(sa, Vm):
  pr = BN // 8
  n_arr = len(sa)

  def body(*refs):
    ins, vv, o = refs[:2 * n_arr], refs[2 * n_arr], refs[2 * n_arr + 1]
    s_lo = sum(ins[2 * a][...] for a in range(n_arr))
    s_hi = sum(ins[2 * a + 1][...] for a in range(n_arr))
    lo_u = jnp.concatenate(
        [s_lo[:, NS * l: NS * (l + 1)] for l in range(8)], axis=0)
    hi_u = jnp.concatenate(
        [s_hi[:, NS * l: NS * (l + 1)] for l in range(8)], axis=0)
    lo_n = jnp.dot(lo_u.T, vv[...], preferred_element_type=jnp.float32)
    hi_n = jnp.dot(hi_u.T, vv[...], preferred_element_type=jnp.float32)
    cnt = jnp.maximum(hi_n[12:13, :], 1.0)
    o[...] = jnp.concatenate([lo_n, hi_n[:12, :]], axis=0) / cnt

  def pspec(pc):
    return pl.BlockSpec((pr, 128), lambda i, o=pc * OS_BLK: (i + o, 0))

  return pl.pallas_call(
      body,
      grid=(OS_BLK,),
      in_specs=[pspec(pc) for _ in range(n_arr) for pc in range(2)]
               + [pl.BlockSpec((BN, BN), lambda i: (0, 0))],
      out_specs=pl.BlockSpec((NOUT, BN), lambda i: (0, i)),
      out_shape=jax.ShapeDtypeStruct((NOUT, N_TOT), jnp.float32),
  )(*[a for a in sa for _ in range(2)], Vm)


def _constants():
  norm = 1.0 / np.sqrt(np.float32(NS))
  perm = np.empty((NWCOL,), np.int64)
  for i in range(NS):
    perm[i * 20: i * 20 + 16] = i * 16 + np.arange(16)
    perm[i * 20 + 16: i * 20 + 20] = NS * NS + i * NV + np.arange(NV)
  Rm = np.zeros((NS, NWCOL), np.float32)
  for i in range(NS):
    Rm[i, i * 20: (i + 1) * 20] = 1.0
  Tm = np.zeros((NWCOL, PAD), np.float32)
  for i in range(NS):
    for cidx in range(16):
      Tm[i * 20 + cidx, cidx] = norm
    for j in range(NV):
      for m in range(3):
        Tm[i * 20 + 16 + j, 16 + 3 * j + m] = norm
  Um = np.zeros((9, PAD), np.float32)
  Um[0, :16] = 1.0
  for j in range(NV):
    for m in range(3):
      Um[1 + m, 16 + 3 * j + m] = 1.0
  Vm = np.zeros((BN, BN), np.float32)
  for l in range(8):
    for r in range(BN // 8):
      Vm[l * (BN // 8) + r, 8 * r + l] = 1.0
  return perm, Rm, Tm, Um, Vm


_PERM, _R, _T, _U, _V = _constants()


def kernel(node_attr, edge_attr, edge_sh, fc_w1, fc_b1, fc_w2, fc_b2,
           edge_index):
  idx32 = edge_index.astype(jnp.int32)
  src, dst = idx32[0], idx32[1]
  w2p = fc_w2[:, _PERM]
  b2p = fc_b2[_PERM].reshape(1, NWCOL)
  b1 = fc_b1.reshape(1, NF)
  Rj, Tj, Uj = jnp.asarray(_R), jnp.asarray(_T), jnp.asarray(_U)
  Vj = jnp.asarray(_V)
  Kc = jnp.zeros((1, PAD), jnp.float32).at[0, NOUT].set(1.0)
  ea_t, sh_t = edge_attr.T, edge_sh.T

  parts = []
  for s in range(SPLITS):
    c0, nch = s * SP_CH, SP_CH
    xsp = _sc_gather(node_attr, dst, c0, nch)
    tp_lo, tp_hi = _tc_dense(ea_t, xsp, sh_t, fc_w1, b1, w2p, b2p,
                             Rj, Tj, Uj, Kc, c0, nch)
    parts.append(_sc_scatter(tp_lo, tp_hi, src, c0, nch))
  packed = [p.reshape(2 * OUT_STRIDE // 8, 128) for p in parts]
  return _tc_mean(packed, Vj).T
